# baseline (speedup 1.0000x reference)
import jax
import jax.numpy as jnp
from jax.experimental import pallas as pl
from jax.experimental.pallas import tpu as pltpu


_ROWS_PER_BLOCK = 512


def _masked_mse_block(x_ref, p_ref, m_ref, sum_ref, cnt_ref):
    i = pl.program_id(0)

    @pl.when(i == 0)
    def _init():
        sum_ref[0, 0] = 0.0
        cnt_ref[0, 0] = 0.0

    m = m_ref[...]
    d = p_ref[...] - x_ref[...]
    part = jnp.sum(d * d * m)
    sum_ref[0, 0] += part
    cnt_ref[0, 0] += jnp.sum(m)


def kernel(x, predictions, mask):
    b, s, d = x.shape
    n = b * s
    xf = x.reshape(n, d)
    pf = predictions.reshape(n, d)
    mf = mask.reshape(n, 1).astype(x.dtype)

    grid = n // _ROWS_PER_BLOCK
    total, cnt = pl.pallas_call(
        _masked_mse_block,
        grid=(grid,),
        in_specs=[
            pl.BlockSpec((_ROWS_PER_BLOCK, d), lambda i: (i, 0)),
            pl.BlockSpec((_ROWS_PER_BLOCK, d), lambda i: (i, 0)),
            pl.BlockSpec((_ROWS_PER_BLOCK, 1), lambda i: (i, 0)),
        ],
        out_specs=[
            pl.BlockSpec(memory_space=pltpu.SMEM),
            pl.BlockSpec(memory_space=pltpu.SMEM),
        ],
        out_shape=[
            jax.ShapeDtypeStruct((1, 1), x.dtype),
            jax.ShapeDtypeStruct((1, 1), x.dtype),
        ],
    )(xf, pf, mf)

    total = total[0, 0]
    cnt = cnt[0, 0] * d
    loss = total / jnp.maximum(cnt, 1.0)
    return jnp.where(cnt == 0, jnp.asarray(0.0, dtype=x.dtype), loss)
